# Initial kernel scaffold; baseline (speedup 1.0000x reference)
import jax
import jax.numpy as jnp
from jax.experimental import pallas as pl


def kernel(x, edge_index, edge_attr, f1_W1, f1_b1, f1_W2, f1_b2, f1_W3, f2_W1, f2_b1, f2_W2, f2_b2, f2_W3, bn1_gamma, bn1_beta, bn2_gamma, bn2_beta):
    raise NotImplementedError("write your pallas kernel here")


# speedup vs baseline: 2.7904x; 2.7904x over previous
import functools

import jax
import jax.numpy as jnp
from jax import lax
from jax.experimental import pallas as pl
from jax.experimental.pallas import tpu as pltpu
from jax.experimental.pallas import tpu_sc as plsc

_N = 10000
_E = 160000
_F = 32
_DE = 4
_H = 64

_NW = 32
_IDXW = 128
_ROWS_W = 40
_EW = _ROWS_W * _IDXW
_EPAD = _NW * _EW
_BLK = 8
_ECH = _BLK * _IDXW
_NCH = _ROWS_W // _BLK

_NPAD = 10016
_DUMMY = 10000
_STRIPE = _NPAD // 16


def _sc_mesh():
    return plsc.VectorSubcoreMesh(core_axis_name="c", subcore_axis_name="s")


def _sc_gather(table, idx2d):
    @functools.partial(
        pl.kernel,
        mesh=_sc_mesh(),
        out_type=jax.ShapeDtypeStruct((_EPAD, _F), jnp.float32),
        scratch_types=[
            pltpu.VMEM((_BLK, _IDXW), jnp.int32),
            pltpu.VMEM((_ECH, _F), jnp.float32),
            pltpu.SemaphoreType.DMA,
        ],
        compiler_params=pltpu.CompilerParams(use_tc_tiling_on_sc=False),
    )
    def k(table_hbm, idx_hbm, out_hbm, idx_v, rows_v, sem):
        wid = lax.axis_index("s") * 2 + lax.axis_index("c")
        row0 = wid * _ROWS_W
        for j in range(_NCH):
            pltpu.sync_copy(idx_hbm.at[pl.ds(row0 + j * _BLK, _BLK)], idx_v)
            cps = [
                pltpu.async_copy(
                    table_hbm.at[idx_v.at[b]],
                    rows_v.at[pl.ds(b * _IDXW, _IDXW)],
                    sem,
                )
                for b in range(_BLK)
            ]
            for c in cps:
                c.wait()
            pltpu.sync_copy(
                rows_v, out_hbm.at[pl.ds(wid * _EW + j * _ECH, _ECH)]
            )

    return k(table, idx2d)


def _sc_scatter(msg, dst2d, z_nodes, z_cnt, ones_cnt, with_cnt):
    outs = [jax.ShapeDtypeStruct((2, _NPAD, _F), jnp.float32)]
    scratch = [
        pltpu.VMEM((_BLK, _IDXW), jnp.int32),
        pltpu.VMEM((_ECH, _F), jnp.float32),
        pltpu.VMEM_SHARED((_NPAD, _F), jnp.float32),
    ]
    if with_cnt:
        outs.append(jax.ShapeDtypeStruct((2, _NPAD, 16), jnp.float32))
        scratch.append(pltpu.VMEM((_IDXW, 16), jnp.float32))
        scratch.append(pltpu.VMEM_SHARED((_NPAD, 16), jnp.float32))

    @functools.partial(
        pl.kernel,
        mesh=_sc_mesh(),
        out_type=tuple(outs) if with_cnt else outs[0],
        scratch_types=scratch,
        compiler_params=pltpu.CompilerParams(use_tc_tiling_on_sc=False),
    )
    def k(msg_hbm, dst_hbm, zn_hbm, zc_hbm, ones_hbm, *refs):
        if with_cnt:
            out_s, out_c, idx_v, msg_v, acc, ones_v, cacc = refs
        else:
            out_s, idx_v, msg_v, acc = refs
        cid = lax.axis_index("c")
        sid = lax.axis_index("s")
        wid = sid * 2 + cid
        stripe = pl.ds(sid * _STRIPE, _STRIPE)
        pltpu.sync_copy(zn_hbm, acc.at[stripe])
        if with_cnt:
            pltpu.sync_copy(zc_hbm, cacc.at[stripe])
            pltpu.sync_copy(ones_hbm, ones_v)
        plsc.subcore_barrier()
        row0 = wid * _ROWS_W
        for j in range(_NCH):
            pltpu.sync_copy(dst_hbm.at[pl.ds(row0 + j * _BLK, _BLK)], idx_v)
            pltpu.sync_copy(
                msg_hbm.at[pl.ds(wid * _EW + j * _ECH, _ECH)], msg_v
            )
            for b in range(_BLK):
                pltpu.sync_copy(
                    msg_v.at[pl.ds(b * _IDXW, _IDXW)],
                    acc.at[idx_v.at[b]],
                    add=True,
                )
                if with_cnt:
                    pltpu.sync_copy(ones_v, cacc.at[idx_v.at[b]], add=True)
        plsc.subcore_barrier()
        pltpu.sync_copy(acc.at[stripe], out_s.at[cid, stripe])
        if with_cnt:
            pltpu.sync_copy(cacc.at[stripe], out_c.at[cid, stripe])

    return k(msg, dst2d, z_nodes, z_cnt, ones_cnt)


_BE = 1024


def _tc_msg(ea, xs, w1t, b1, w2t, b2, w3t, smat):
    grid = (_EPAD // _BE,)

    def body(ea_ref, xs_ref, w1_ref, b1_ref, w2_ref, b2_ref, w3_ref,
             s_ref, out_ref):
        h = jnp.dot(ea_ref[...], w1_ref[...],
                    preferred_element_type=jnp.float32) + b1_ref[...]
        h = jnp.maximum(h, 0.0)
        h = jnp.dot(h, w2_ref[...],
                    preferred_element_type=jnp.float32) + b2_ref[...]
        h = jnp.maximum(h, 0.0)
        th = jnp.dot(h, w3_ref[...], preferred_element_type=jnp.float32)
        xst = jnp.concatenate([xs_ref[...]] * _F, axis=1)
        out_ref[...] = jnp.dot(th * xst, s_ref[...],
                               preferred_element_type=jnp.float32)

    return pl.pallas_call(
        body,
        grid=grid,
        in_specs=[
            pl.BlockSpec((_BE, _DE), lambda i: (i, 0)),
            pl.BlockSpec((_BE, _F), lambda i: (i, 0)),
            pl.BlockSpec((_DE, _H), lambda i: (0, 0)),
            pl.BlockSpec((1, _H), lambda i: (0, 0)),
            pl.BlockSpec((_H, _H), lambda i: (0, 0)),
            pl.BlockSpec((1, _H), lambda i: (0, 0)),
            pl.BlockSpec((_H, _F * _F), lambda i: (0, 0)),
            pl.BlockSpec((_F * _F, _F), lambda i: (0, 0)),
        ],
        out_specs=pl.BlockSpec((_BE, _F), lambda i: (i, 0)),
        out_shape=jax.ShapeDtypeStruct((_EPAD, _F), jnp.float32),
    )(ea, xs, w1t, b1, w2t, b2, w3t, smat)


def _tc_finalize(psums, pcnt, gamma, beta, resid):
    n_in = 4 if resid is None else 5

    def body(*refs):
        if resid is None:
            ps_ref, pc_ref, g_ref, b_ref, out_ref = refs
        else:
            ps_ref, pc_ref, g_ref, b_ref, r_ref, out_ref = refs
        s = ps_ref[0, 0:_N, :] + ps_ref[1, 0:_N, :]
        c = pc_ref[0, 0:_N, 0:1] + pc_ref[1, 0:_N, 0:1]
        h = s / jnp.maximum(c, 1.0)
        mu = jnp.mean(h, axis=0, keepdims=True)
        xc = h - mu
        var = jnp.mean(xc * xc, axis=0, keepdims=True)
        y = xc * lax.rsqrt(var + 1e-5) * g_ref[...] + b_ref[...]
        if resid is not None:
            y = y + r_ref[...]
        out_ref[...] = jnp.maximum(y, 0.0)

    args = [psums, pcnt, gamma.reshape(1, _F), beta.reshape(1, _F)]
    if resid is not None:
        args.append(resid)
    return pl.pallas_call(
        body,
        out_shape=jax.ShapeDtypeStruct((_N, _F), jnp.float32),
    )(*args)


def _prep_w3(w3):
    return w3.reshape(_F, _F, _H).transpose(1, 0, 2).reshape(_F * _F, _H).T


def kernel(x, edge_index, edge_attr, f1_W1, f1_b1, f1_W2, f1_b2, f1_W3,
           f2_W1, f2_b1, f2_W2, f2_b2, f2_W3, bn1_gamma, bn1_beta,
           bn2_gamma, bn2_beta):
    src = edge_index[0]
    dst = edge_index[1]
    npad = _EPAD - _E
    src2d = jnp.concatenate(
        [src, jnp.zeros((npad,), jnp.int32)]).reshape(-1, _IDXW)
    dst2d = jnp.concatenate(
        [dst, jnp.full((npad,), _DUMMY, jnp.int32)]).reshape(-1, _IDXW)
    ea_p = jnp.concatenate(
        [edge_attr, jnp.zeros((npad, _DE), jnp.float32)], axis=0)

    z_nodes = jnp.zeros((_STRIPE, _F), jnp.float32)
    z_cnt = jnp.zeros((_STRIPE, 16), jnp.float32)
    ones_cnt = jnp.ones((_IDXW, 16), jnp.float32)
    smat = jnp.repeat(jnp.eye(_F, dtype=jnp.float32), _F, axis=0)

    xs1 = _sc_gather(x, src2d)
    msg1 = _tc_msg(ea_p, xs1, f1_W1.T, f1_b1.reshape(1, _H), f1_W2.T,
                   f1_b2.reshape(1, _H), _prep_w3(f1_W3), smat)
    ps1, pc = _sc_scatter(msg1, dst2d, z_nodes, z_cnt, ones_cnt, True)
    h = _tc_finalize(ps1, pc, bn1_gamma, bn1_beta, None)

    xs2 = _sc_gather(h, src2d)
    msg2 = _tc_msg(ea_p, xs2, f2_W1.T, f2_b1.reshape(1, _H), f2_W2.T,
                   f2_b2.reshape(1, _H), _prep_w3(f2_W3), smat)
    ps2 = _sc_scatter(msg2, dst2d, z_nodes, z_cnt, ones_cnt, False)
    return _tc_finalize(ps2, pc, bn2_gamma, bn2_beta, x)
